# Initial kernel scaffold; baseline (speedup 1.0000x reference)
import jax
import jax.numpy as jnp
from jax.experimental import pallas as pl


def kernel(x, edge_index, H1_w, H1_b, W1_w, W1_b, H2_w, H2_b, W2_w, W2_b):
    raise NotImplementedError("write your pallas kernel here")


# speedup vs baseline: 43.6053x; 43.6053x over previous
import functools

import jax
import jax.numpy as jnp
from jax import lax
from jax.experimental import pallas as pl
from jax.experimental.pallas import tpu as pltpu
from jax.experimental.pallas import tpu_sc as plsc

N = 50000
FP = 1778
E = 3200000

NW = 32
CHUNK = 2048
NCHUNK = 49
E_PT = CHUNK * NCHUNK
E_PAD = E_PT * NW
N_PAD = 50176
SLICE = N_PAD // 16

ROWS = CHUNK // 128

_mesh = plsc.VectorSubcoreMesh(core_axis_name="c", subcore_axis_name="s")


@functools.partial(
    pl.kernel,
    mesh=_mesh,
    out_type=jax.ShapeDtypeStruct((4 * N_PAD,), jnp.float32),
    compiler_params=pltpu.CompilerParams(needs_layout_passes=False),
    scratch_types=[
        pltpu.VMEM((N_PAD,), jnp.float32),
        pltpu.VMEM((N_PAD,), jnp.float32),
        pltpu.VMEM((CHUNK,), jnp.int32),
        pltpu.VMEM((ROWS, 128), jnp.int32),
        pltpu.VMEM((ROWS, 128), jnp.float32),
        pltpu.VMEM((ROWS, 128), jnp.float32),
        pltpu.VMEM((SLICE,), jnp.float32),
        pltpu.VMEM_SHARED((N_PAD,), jnp.float32),
        pltpu.VMEM_SHARED((N_PAD,), jnp.float32),
        pltpu.SemaphoreType.DMA,
    ],
)
def _segment_sum_sc(xx, srch, dsth, out, x0v, x1v, srcv, dstv, v0v, v1v,
                    zbv, acc0, acc1, sem):
    cid = lax.axis_index("c")
    sid = lax.axis_index("s")
    wid = sid * 2 + cid

    pltpu.sync_copy(xx.at[pl.ds(0, N_PAD)], x0v)
    pltpu.sync_copy(xx.at[pl.ds(N_PAD, N_PAD)], x1v)

    def _z(i, carry):
        zbv[pl.ds(i * 16, 16)] = jnp.zeros((16,), jnp.float32)
        return carry
    lax.fori_loop(0, SLICE // 16, _z, 0)
    pltpu.sync_copy(zbv, acc0.at[pl.ds(sid * SLICE, SLICE)])
    pltpu.sync_copy(zbv, acc1.at[pl.ds(sid * SLICE, SLICE)])
    plsc.subcore_barrier()

    def _chunk(k, carry):
        base = wid * E_PT + k * CHUNK
        row0 = wid * (E_PT // 128) + k * ROWS
        pltpu.sync_copy(srch.at[pl.ds(base, CHUNK)], srcv)
        pltpu.sync_copy(dsth.at[pl.ds(row0, ROWS)], dstv)

        def _g(i, c2):
            s16 = srcv[pl.ds(i * 16, 16)]
            r = i // 8
            col = (i % 8) * 16
            v0v[r, pl.ds(col, 16)] = plsc.load_gather(x0v, [s16])
            v1v[r, pl.ds(col, 16)] = plsc.load_gather(x1v, [s16])
            return c2
        lax.fori_loop(0, CHUNK // 16, _g, 0)

        cps = []
        for j in range(ROWS):
            cps.append(pltpu.async_copy(
                v0v.at[j], acc0.at[dstv.at[j]], sem, add=True))
            cps.append(pltpu.async_copy(
                v1v.at[j], acc1.at[dstv.at[j]], sem, add=True))
        for cp in cps:
            cp.wait()
        return carry
    lax.fori_loop(0, NCHUNK, _chunk, 0)

    plsc.subcore_barrier()
    pltpu.sync_copy(acc0.at[pl.ds(sid * SLICE, SLICE)], zbv)
    pltpu.sync_copy(zbv, out.at[pl.ds(cid * 2 * N_PAD + sid * SLICE, SLICE)])
    pltpu.sync_copy(acc1.at[pl.ds(sid * SLICE, SLICE)], zbv)
    pltpu.sync_copy(zbv, out.at[pl.ds((cid * 2 + 1) * N_PAD + sid * SLICE, SLICE)])


def _affine_body(p_ref, xx_ref, hw_ref, hb_ref, o_ref):
    v = p_ref[0] + p_ref[1] + xx_ref[...]
    hw = hw_ref[...]
    z = (hw[:, 0:1] * v[0:1, :] + hw[:, 1:2] * v[1:2, :]) + hb_ref[...]
    o_ref[...] = 1.0 / (1.0 + jnp.exp(-z))


def _affine_sigmoid(p, xx, H_w, H_b):
    return pl.pallas_call(
        _affine_body,
        out_shape=jax.ShapeDtypeStruct((2, N_PAD), jnp.float32),
    )(p, xx, H_w, H_b.reshape(2, 1))


R = 1000


def _fp_body(a1t_ref, p2t_ref, h2wt_ref, h2b_ref, w1t_ref, b1_ref,
             w2t_ref, b2_ref, o_ref):
    a1t = a1t_ref[...]
    v2t = p2t_ref[:, 0, :] + p2t_ref[:, 1, :] + a1t
    z2 = jnp.dot(v2t, h2wt_ref[...],
                 preferred_element_type=jnp.float32) + h2b_ref[...]
    a2t = 1.0 / (1.0 + jnp.exp(-z2))

    def _soft(at, wt_ref, b_ref):
        l = (at[:, 0:1] * wt_ref[0:1, :] + at[:, 1:2] * wt_ref[1:2, :]
             + b_ref[...])
        m = jnp.max(l, axis=1, keepdims=True)
        e = jnp.exp(l - m)
        return e / jnp.sum(e, axis=1, keepdims=True)

    o_ref[...] = _soft(a1t, w1t_ref, b1_ref) + _soft(a2t, w2t_ref, b2_ref)


def _fingerprint(a1t, p2t, H2_w, H2_b, W1_w, W1_b, W2_w, W2_b):
    grid = (N // R,)
    return pl.pallas_call(
        _fp_body,
        grid=grid,
        in_specs=[
            pl.BlockSpec((R, 2), lambda i: (i, 0)),
            pl.BlockSpec((R, 2, 2), lambda i: (i, 0, 0)),
            pl.BlockSpec((2, 2), lambda i: (0, 0)),
            pl.BlockSpec((1, 2), lambda i: (0, 0)),
            pl.BlockSpec((2, FP), lambda i: (0, 0)),
            pl.BlockSpec((1, FP), lambda i: (0, 0)),
            pl.BlockSpec((2, FP), lambda i: (0, 0)),
            pl.BlockSpec((1, FP), lambda i: (0, 0)),
        ],
        out_specs=pl.BlockSpec((R, FP), lambda i: (i, 0)),
        out_shape=jax.ShapeDtypeStruct((N, FP), jnp.float32),
    )(a1t, p2t, H2_w.T, H2_b.reshape(1, 2), W1_w.T, W1_b.reshape(1, FP),
      W2_w.T, W2_b.reshape(1, FP))


def kernel(x, edge_index, H1_w, H1_b, W1_w, W1_b, H2_w, H2_b, W2_w, W2_b):
    ei = edge_index.astype(jnp.int32)
    npad = E_PAD - E
    src = jnp.concatenate([ei[0], jnp.zeros((npad,), jnp.int32)])
    dst = jnp.concatenate(
        [ei[1], N + (jnp.arange(npad, dtype=jnp.int32) % (N_PAD - N))])
    dst2d = dst.reshape(E_PAD // 128, 128)

    xx = jnp.zeros((2, N_PAD), jnp.float32).at[:, :N].set(x.T)

    p1 = _segment_sum_sc(xx.reshape(-1), src, dst2d).reshape(2, 2, N_PAD)
    a1 = _affine_sigmoid(p1, xx, H1_w, H1_b)
    p2 = _segment_sum_sc(a1.reshape(-1), src, dst2d).reshape(2, 2, N_PAD)

    a1t = a1.T[:N]
    p2t = p2.transpose(2, 0, 1)[:N]
    return _fingerprint(a1t, p2t, H2_w, H2_b, W1_w, W1_b, W2_w, W2_b)
